# baseline (speedup 1.0000x reference)
import functools

import jax
import jax.numpy as jnp
from jax import lax
from jax.experimental import pallas as pl
from jax.experimental.pallas import tpu as pltpu
from jax.experimental.pallas import tpu_sc as plsc

D_MODEL = 2048
LANES = 16
NUM_WORKERS = 32
K = 8
NB = 2


def _pack_table(table):
    bf = table.astype(jnp.bfloat16).reshape(table.shape[0], -1, 2, LANES)
    bits = lax.bitcast_convert_type(bf, jnp.uint16).astype(jnp.uint32)
    words = bits[:, :, 0, :] | (bits[:, :, 1, :] << 16)
    return lax.bitcast_convert_type(words, jnp.int32).reshape(
        table.shape[0], table.shape[1] // 2)


@jax.jit
def _gather_add(seq_tok, struct_tok, seq_packed, struct_packed):
    n = seq_tok.shape[0]
    v_seq = seq_packed.shape[0]
    per_w = n // NUM_WORKERS
    n_chunks = per_w // K
    n_outer = n_chunks // NB
    mesh = plsc.VectorSubcoreMesh(core_axis_name="c", subcore_axis_name="s")

    @functools.partial(
        pl.kernel,
        mesh=mesh,
        compiler_params=pltpu.CompilerParams(needs_layout_passes=False),
        out_type=jax.ShapeDtypeStruct((n, D_MODEL), jnp.float32),
        scratch_types=[
            pltpu.VMEM((per_w + LANES,), jnp.int32),
            pltpu.VMEM((per_w,), jnp.int32),
            pltpu.VMEM((v_seq, D_MODEL // 2), jnp.int32),
            pltpu.VMEM((NB, K, D_MODEL // 2), jnp.int32),
            pltpu.VMEM((NB, K, D_MODEL), jnp.float32),
            pltpu.SemaphoreType.DMA,
            pltpu.SemaphoreType.DMA,
            pltpu.SemaphoreType.DMA,
            pltpu.SemaphoreType.DMA,
        ],
    )
    def k(seq_tok_hbm, struct_tok_hbm, seq_pk_hbm, struct_pk_hbm, out_hbm,
          sidx, tidx, seq_pk, buf_g, buf_w,
          sem_g0, sem_g1, sem_w0, sem_w1):
        sem_g = (sem_g0, sem_g1)
        sem_w = (sem_w0, sem_w1)
        wid = lax.axis_index("s") * 2 + lax.axis_index("c")
        base = wid * per_w
        pltpu.sync_copy(seq_tok_hbm.at[pl.ds(base, per_w)],
                        sidx.at[pl.ds(0, per_w)])
        pltpu.sync_copy(struct_tok_hbm.at[pl.ds(base, per_w)], tidx)
        pltpu.sync_copy(seq_pk_hbm, seq_pk)

        def gather(off, b):
            pltpu.async_copy(
                struct_pk_hbm.at[tidx.at[pl.ds(off, K)]], buf_g.at[b],
                sem_g[b])

        def wait_gather(off, b):
            pltpu.make_async_copy(
                struct_pk_hbm.at[tidx.at[pl.ds(off, K)]], buf_g.at[b],
                sem_g[b]).wait()

        def writeback(off, b):
            return pltpu.make_async_copy(
                buf_w.at[b], out_hbm.at[pl.ds(base + off, K)], sem_w[b])

        lanes_iota = lax.iota(jnp.int32, LANES)
        himask = jnp.full((LANES,), -65536, jnp.int32)

        gather(0, 0)
        gather(K, 1)

        def outer(o, _):
            for b in range(NB):
                g = o * NB + b
                off = g * K
                wait_gather(off, b)
                @pl.when(o > 0)
                def _():
                    writeback(off - NB * K, b).wait()
                rows16 = sidx[pl.ds(off, LANES)]
                for i in range(K):
                    splat = jnp.take_along_axis(
                        rows16, jnp.full((LANES,), i, jnp.int32), axis=0)

                    @plsc.parallel_loop(0, D_MODEL, 2 * LANES, unroll=4)
                    def _(j, b=b, i=i, splat=splat):
                        wq = plsc.load_gather(
                            seq_pk,
                            [splat, lanes_iota + lax.shift_right_logical(j, 1)])
                        ws = buf_g[b, i, pl.ds(lax.shift_right_logical(j, 1),
                                               LANES)]
                        lo = (plsc.bitcast(lax.shift_left(wq, 16), jnp.float32)
                              + plsc.bitcast(lax.shift_left(ws, 16),
                                             jnp.float32))
                        hi = (plsc.bitcast(wq & himask, jnp.float32)
                              + plsc.bitcast(ws & himask, jnp.float32))
                        buf_w[b, i, pl.ds(j, LANES)] = lo
                        buf_w[b, i, pl.ds(j + LANES, LANES)] = hi
                writeback(off, b).start()
                @pl.when(o < n_outer - 1)
                def _():
                    gather(off + NB * K, b)
            return 0

        lax.fori_loop(0, n_outer, outer, 0)

        for b in range(NB):
            off = (n_chunks - NB + b) * K
            writeback(off, b).wait()

    return k(seq_tok, struct_tok, seq_packed, struct_packed)


def kernel(sequence_tokens, structure_tokens, seq_table, struct_table):
    b, s = sequence_tokens.shape
    n = b * s
    seq_tok = sequence_tokens.reshape(n).astype(jnp.int32)
    struct_tok = structure_tokens.reshape(n).astype(jnp.int32)
    out = _gather_add(seq_tok, struct_tok, _pack_table(seq_table),
                      _pack_table(struct_table))
    return out.reshape(b, s, D_MODEL)


# speedup vs baseline: 1.3715x; 1.3715x over previous
import functools

import jax
import jax.numpy as jnp
from jax import lax
from jax.experimental import pallas as pl
from jax.experimental.pallas import tpu as pltpu
from jax.experimental.pallas import tpu_sc as plsc

D_MODEL = 2048
LANES = 16
NUM_WORKERS = 32
K = 8
NB = 2


def _pack_table(table):
    half = table.shape[1] // 2
    bf = table.astype(jnp.bfloat16)
    lo = lax.bitcast_convert_type(bf[:, :half], jnp.uint16).astype(jnp.uint32)
    hi = lax.bitcast_convert_type(bf[:, half:], jnp.uint16).astype(jnp.uint32)
    return lax.bitcast_convert_type(lo | (hi << 16), jnp.int32)


@jax.jit
def _gather_add(seq_tok, struct_tok, seq_packed, struct_packed):
    n = seq_tok.shape[0]
    v_seq = seq_packed.shape[0]
    per_w = n // NUM_WORKERS
    n_chunks = per_w // K
    n_outer = n_chunks // NB
    mesh = plsc.VectorSubcoreMesh(core_axis_name="c", subcore_axis_name="s")

    @functools.partial(
        pl.kernel,
        mesh=mesh,
        compiler_params=pltpu.CompilerParams(needs_layout_passes=False),
        out_type=jax.ShapeDtypeStruct((n, D_MODEL), jnp.float32),
        scratch_types=[
            pltpu.VMEM((per_w + LANES,), jnp.int32),
            pltpu.VMEM((per_w,), jnp.int32),
            pltpu.VMEM((v_seq, D_MODEL // 2), jnp.int32),
            pltpu.VMEM((NB, K, D_MODEL // 2), jnp.int32),
            pltpu.VMEM((NB, K, D_MODEL), jnp.float32),
            pltpu.SemaphoreType.DMA,
            pltpu.SemaphoreType.DMA,
            pltpu.SemaphoreType.DMA,
            pltpu.SemaphoreType.DMA,
        ],
    )
    def k(seq_tok_hbm, struct_tok_hbm, seq_pk_hbm, struct_pk_hbm, out_hbm,
          sidx, tidx, seq_pk, buf_g, buf_w,
          sem_g0, sem_g1, sem_w0, sem_w1):
        sem_g = (sem_g0, sem_g1)
        sem_w = (sem_w0, sem_w1)
        wid = lax.axis_index("s") * 2 + lax.axis_index("c")
        base = wid * per_w
        pltpu.sync_copy(seq_tok_hbm.at[pl.ds(base, per_w)],
                        sidx.at[pl.ds(0, per_w)])
        pltpu.sync_copy(struct_tok_hbm.at[pl.ds(base, per_w)], tidx)
        pltpu.sync_copy(seq_pk_hbm, seq_pk)

        def gather(off, b):
            pltpu.async_copy(
                struct_pk_hbm.at[tidx.at[pl.ds(off, K)]], buf_g.at[b],
                sem_g[b])

        def wait_gather(off, b):
            pltpu.make_async_copy(
                struct_pk_hbm.at[tidx.at[pl.ds(off, K)]], buf_g.at[b],
                sem_g[b]).wait()

        def writeback(off, b):
            return pltpu.make_async_copy(
                buf_w.at[b], out_hbm.at[pl.ds(base + off, K)], sem_w[b])

        lanes_iota = lax.iota(jnp.int32, LANES)
        himask = jnp.full((LANES,), -65536, jnp.int32)

        gather(0, 0)
        gather(K, 1)

        def outer(o, _):
            for b in range(NB):
                g = o * NB + b
                off = g * K
                wait_gather(off, b)
                @pl.when(o > 0)
                def _():
                    writeback(off - NB * K, b).wait()
                rows16 = sidx[pl.ds(off, LANES)]
                for i in range(K):
                    splat = jnp.take_along_axis(
                        rows16, jnp.full((LANES,), i, jnp.int32), axis=0)

                    @plsc.parallel_loop(0, D_MODEL // 2, LANES, unroll=4)
                    def _(j, b=b, i=i, splat=splat):
                        wq = plsc.load_gather(seq_pk, [splat, lanes_iota + j])
                        ws = buf_g[b, i, pl.ds(j, LANES)]
                        lo = (plsc.bitcast(lax.shift_left(wq, 16), jnp.float32)
                              + plsc.bitcast(lax.shift_left(ws, 16),
                                             jnp.float32))
                        hi = (plsc.bitcast(wq & himask, jnp.float32)
                              + plsc.bitcast(ws & himask, jnp.float32))
                        buf_w[b, i, pl.ds(j, LANES)] = lo
                        buf_w[b, i, pl.ds(j + D_MODEL // 2, LANES)] = hi
                writeback(off, b).start()
                @pl.when(o < n_outer - 1)
                def _():
                    gather(off + NB * K, b)
            return 0

        lax.fori_loop(0, n_outer, outer, 0)

        for b in range(NB):
            off = (n_chunks - NB + b) * K
            writeback(off, b).wait()

    return k(seq_tok, struct_tok, seq_packed, struct_packed)


def kernel(sequence_tokens, structure_tokens, seq_table, struct_table):
    b, s = sequence_tokens.shape
    n = b * s
    seq_tok = sequence_tokens.reshape(n).astype(jnp.int32)
    struct_tok = structure_tokens.reshape(n).astype(jnp.int32)
    out = _gather_add(seq_tok, struct_tok, _pack_table(seq_table),
                      _pack_table(struct_table))
    return out.reshape(b, s, D_MODEL)
